# baseline (speedup 1.0000x reference)
import functools

import jax
import jax.numpy as jnp
import numpy as np
from jax.experimental import pallas as pl
from jax.experimental.pallas import tpu as pltpu

N = 4096
BM = 256
CW = 1024
NC = N // CW
RG = CW // BM


def _scaled_rhs_kernel(x_ref, w_ref, wb_ref, out_ref):
    z = jnp.dot(x_ref[...], w_ref[...], preferred_element_type=jnp.float32)
    out_ref[0] = wb_ref[0, 0] * z
    out_ref[1] = wb_ref[1, 0] * z


def _main_kernel(a_ref, zs_ref, w2_ref, wb_ref, b1_ref,
                 u1_ref, u2p_ref, vs_out_ref, vs_scr):
    r = pl.program_id(0)

    @pl.when(r == 0)
    def _zero_scratch():
        vs_scr[...] = jnp.zeros_like(vs_scr)

    acc = (
        jnp.dot(a_ref[0], zs_ref[0], preferred_element_type=jnp.float32)
        + jnp.dot(a_ref[1], zs_ref[1], preferred_element_type=jnp.float32)
    )
    u1 = jnp.maximum(acc + b1_ref[...], 0.0)
    u1_ref[...] = u1
    v = jnp.dot(u1, w2_ref[...], preferred_element_type=jnp.float32)
    v0 = wb_ref[0, 0] * v
    v1 = wb_ref[1, 0] * v
    vs_scr[0, pl.ds(r * BM, BM), :] = v0
    vs_scr[1, pl.ds(r * BM, BM), :] = v1
    vs_out_ref[0] = v0
    vs_out_ref[1] = v1

    u2p = jnp.zeros_like(u2p_ref)
    for c in range(NC - 1):
        sl = slice(c * CW, (c + 1) * CW)
        mask = (c < r // RG).astype(jnp.float32)
        u2p += mask * (
            jnp.dot(a_ref[0][:, sl], vs_scr[0, sl, :],
                    preferred_element_type=jnp.float32)
            + jnp.dot(a_ref[1][:, sl], vs_scr[1, sl, :],
                      preferred_element_type=jnp.float32)
        )
    u2p_ref[...] = u2p


def _remainder_kernel(rmap_ref, cmap_ref, a_ref, vs_ref, u1_ref, u2p_ref,
                      b2_ref, out_ref, acc_ref):
    t = pl.program_id(0)
    r = rmap_ref[t]
    c = cmap_ref[t]

    @pl.when(c == r)
    def _init():
        acc_ref[...] = u2p_ref[...]

    acc_ref[...] += (
        jnp.dot(a_ref[0], vs_ref[0], preferred_element_type=jnp.float32)
        + jnp.dot(a_ref[1], vs_ref[1], preferred_element_type=jnp.float32)
    )

    @pl.when(c == NC - 1)
    def _final():
        out_ref[...] = 0.5 * (u1_ref[...] + acc_ref[...] + b2_ref[...])


@jax.jit
def kernel(feature, A, W1, b1, W2, b2, weight_b):
    n = A.shape[1]
    hid = W1.shape[1]
    out_dim = W2.shape[1]

    zs = pl.pallas_call(
        _scaled_rhs_kernel,
        out_shape=jax.ShapeDtypeStruct((2, n, hid), jnp.float32),
    )(feature, W1, weight_b)

    u1, u2p, vs = pl.pallas_call(
        _main_kernel,
        grid=(n // BM,),
        in_specs=[
            pl.BlockSpec((2, BM, n), lambda r: (0, r, 0)),
            pl.BlockSpec((2, n, hid), lambda r: (0, 0, 0)),
            pl.BlockSpec((hid, out_dim), lambda r: (0, 0)),
            pl.BlockSpec((2, 1), lambda r: (0, 0)),
            pl.BlockSpec((1, hid), lambda r: (0, 0)),
        ],
        out_specs=[
            pl.BlockSpec((BM, hid), lambda r: (r, 0)),
            pl.BlockSpec((BM, out_dim), lambda r: (r, 0)),
            pl.BlockSpec((2, BM, out_dim), lambda r: (0, r, 0)),
        ],
        out_shape=[
            jax.ShapeDtypeStruct((n, hid), jnp.float32),
            jax.ShapeDtypeStruct((n, out_dim), jnp.float32),
            jax.ShapeDtypeStruct((2, n, out_dim), jnp.float32),
        ],
        scratch_shapes=[
            pltpu.VMEM((2, n, out_dim), jnp.float32),
        ],
    )(A, zs, W2, weight_b, b1.reshape(1, hid))

    nc = n // CW
    pairs = [(r, c) for r in range(nc) for c in range(r, nc)]
    rmap = jnp.asarray(np.array([p[0] for p in pairs], dtype=np.int32))
    cmap = jnp.asarray(np.array([p[1] for p in pairs], dtype=np.int32))

    grid_spec = pltpu.PrefetchScalarGridSpec(
        num_scalar_prefetch=2,
        grid=(len(pairs),),
        in_specs=[
            pl.BlockSpec((2, CW, CW), lambda t, rm, cm: (0, rm[t], cm[t])),
            pl.BlockSpec((2, CW, out_dim), lambda t, rm, cm: (0, cm[t], 0)),
            pl.BlockSpec((CW, hid), lambda t, rm, cm: (rm[t], 0)),
            pl.BlockSpec((CW, out_dim), lambda t, rm, cm: (rm[t], 0)),
            pl.BlockSpec((1, out_dim), lambda t, rm, cm: (0, 0)),
        ],
        out_specs=pl.BlockSpec((CW, out_dim), lambda t, rm, cm: (rm[t], 0)),
        scratch_shapes=[pltpu.VMEM((CW, out_dim), jnp.float32)],
    )

    out = pl.pallas_call(
        _remainder_kernel,
        grid_spec=grid_spec,
        out_shape=jax.ShapeDtypeStruct((n, out_dim), jnp.float32),
    )(rmap, cmap, A, vs, u1, u2p, b2.reshape(1, out_dim))

    return out


# speedup vs baseline: 1.0363x; 1.0363x over previous
import functools

import jax
import jax.numpy as jnp
import numpy as np
from jax.experimental import pallas as pl
from jax.experimental.pallas import tpu as pltpu

N = 4096
BM = 256
CW = 1024
NC = N // CW
RG = CW // BM


def _scaled_rhs_kernel(x_ref, w_ref, wb_ref, out_ref):
    z = jnp.dot(x_ref[...], w_ref[...], preferred_element_type=jnp.float32)
    out_ref[0] = wb_ref[0, 0] * z
    out_ref[1] = wb_ref[1, 0] * z


def _main_kernel(a_ref, zs_ref, w2_ref, wb_ref, b1_ref,
                 u1_ref, u2p_ref, vs_out_ref, vs_stage, vs_group):
    r = pl.program_id(0)

    @pl.when(r == 0)
    def _zero_group():
        vs_group[...] = jnp.zeros_like(vs_group)

    acc = (
        jnp.dot(a_ref[0], zs_ref[0], preferred_element_type=jnp.float32)
        + jnp.dot(a_ref[1], zs_ref[1], preferred_element_type=jnp.float32)
    )
    u1 = jnp.maximum(acc + b1_ref[...], 0.0)
    u1_ref[...] = u1
    v = jnp.dot(u1, w2_ref[...], preferred_element_type=jnp.float32)
    v0 = wb_ref[0, 0] * v
    v1 = wb_ref[1, 0] * v
    vs_stage[0, pl.ds(r * BM, BM), :] = v0
    vs_stage[1, pl.ds(r * BM, BM), :] = v1
    vs_out_ref[0] = v0
    vs_out_ref[1] = v1

    u2p_ref[...] = (
        jnp.dot(a_ref[0], vs_group[0], preferred_element_type=jnp.float32)
        + jnp.dot(a_ref[1], vs_group[1], preferred_element_type=jnp.float32)
    )

    @pl.when(r % RG == RG - 1)
    def _publish_group():
        g = r // RG
        vs_group[0, pl.ds(g * CW, CW), :] = vs_stage[0, pl.ds(g * CW, CW), :]
        vs_group[1, pl.ds(g * CW, CW), :] = vs_stage[1, pl.ds(g * CW, CW), :]


def _remainder_kernel(rmap_ref, cmap_ref, a_ref, vs_ref, u1_ref, u2p_ref,
                      b2_ref, out_ref, acc_ref):
    t = pl.program_id(0)
    r = rmap_ref[t]
    c = cmap_ref[t]

    @pl.when(c == r)
    def _init():
        acc_ref[...] = u2p_ref[...]

    acc_ref[...] += (
        jnp.dot(a_ref[0], vs_ref[0], preferred_element_type=jnp.float32)
        + jnp.dot(a_ref[1], vs_ref[1], preferred_element_type=jnp.float32)
    )

    @pl.when(c == NC - 1)
    def _final():
        out_ref[...] = 0.5 * (u1_ref[...] + acc_ref[...] + b2_ref[...])


@jax.jit
def kernel(feature, A, W1, b1, W2, b2, weight_b):
    n = A.shape[1]
    hid = W1.shape[1]
    out_dim = W2.shape[1]

    zs = pl.pallas_call(
        _scaled_rhs_kernel,
        out_shape=jax.ShapeDtypeStruct((2, n, hid), jnp.float32),
    )(feature, W1, weight_b)

    u1, u2p, vs = pl.pallas_call(
        _main_kernel,
        grid=(n // BM,),
        in_specs=[
            pl.BlockSpec((2, BM, n), lambda r: (0, r, 0)),
            pl.BlockSpec((2, n, hid), lambda r: (0, 0, 0)),
            pl.BlockSpec((hid, out_dim), lambda r: (0, 0)),
            pl.BlockSpec((2, 1), lambda r: (0, 0)),
            pl.BlockSpec((1, hid), lambda r: (0, 0)),
        ],
        out_specs=[
            pl.BlockSpec((BM, hid), lambda r: (r, 0)),
            pl.BlockSpec((BM, out_dim), lambda r: (r, 0)),
            pl.BlockSpec((2, BM, out_dim), lambda r: (0, r, 0)),
        ],
        out_shape=[
            jax.ShapeDtypeStruct((n, hid), jnp.float32),
            jax.ShapeDtypeStruct((n, out_dim), jnp.float32),
            jax.ShapeDtypeStruct((2, n, out_dim), jnp.float32),
        ],
        scratch_shapes=[
            pltpu.VMEM((2, n, out_dim), jnp.float32),
            pltpu.VMEM((2, n, out_dim), jnp.float32),
        ],
    )(A, zs, W2, weight_b, b1.reshape(1, hid))

    nc = n // CW
    pairs = [(r, c) for r in range(nc) for c in range(r, nc)]
    rmap = jnp.asarray(np.array([p[0] for p in pairs], dtype=np.int32))
    cmap = jnp.asarray(np.array([p[1] for p in pairs], dtype=np.int32))

    grid_spec = pltpu.PrefetchScalarGridSpec(
        num_scalar_prefetch=2,
        grid=(len(pairs),),
        in_specs=[
            pl.BlockSpec((2, CW, CW), lambda t, rm, cm: (0, rm[t], cm[t])),
            pl.BlockSpec((2, CW, out_dim), lambda t, rm, cm: (0, cm[t], 0)),
            pl.BlockSpec((CW, hid), lambda t, rm, cm: (rm[t], 0)),
            pl.BlockSpec((CW, out_dim), lambda t, rm, cm: (rm[t], 0)),
            pl.BlockSpec((1, out_dim), lambda t, rm, cm: (0, 0)),
        ],
        out_specs=pl.BlockSpec((CW, out_dim), lambda t, rm, cm: (rm[t], 0)),
        scratch_shapes=[pltpu.VMEM((CW, out_dim), jnp.float32)],
    )

    out = pl.pallas_call(
        _remainder_kernel,
        grid_spec=grid_spec,
        out_shape=jax.ShapeDtypeStruct((n, out_dim), jnp.float32),
    )(rmap, cmap, A, vs, u1, u2p, b2.reshape(1, out_dim))

    return out


# speedup vs baseline: 1.0669x; 1.0295x over previous
import functools

import jax
import jax.numpy as jnp
import numpy as np
from jax.experimental import pallas as pl
from jax.experimental.pallas import tpu as pltpu

N = 4096
BM = 256
CW = 1024
NC = N // CW
RG = CW // BM


def _scaled_rhs_kernel(x_ref, w_ref, wb_ref, out_ref):
    z = jnp.dot(x_ref[...], w_ref[...], preferred_element_type=jnp.float32)
    out_ref[0] = (wb_ref[0, 0] * z).astype(jnp.bfloat16)
    out_ref[1] = (wb_ref[1, 0] * z).astype(jnp.bfloat16)


def _main_kernel(a_ref, zs_ref, w2_ref, wb_ref, b1_ref,
                 u1_ref, u2p_ref, vs_out_ref, vs_stage, vs_group):
    r = pl.program_id(0)

    @pl.when(r == 0)
    def _zero_group():
        vs_group[...] = jnp.zeros_like(vs_group)

    a0 = a_ref[0].astype(jnp.bfloat16)
    a1 = a_ref[1].astype(jnp.bfloat16)
    acc = (
        jnp.dot(a0, zs_ref[0], preferred_element_type=jnp.float32)
        + jnp.dot(a1, zs_ref[1], preferred_element_type=jnp.float32)
    )
    u1 = jnp.maximum(acc + b1_ref[...], 0.0)
    u1_ref[...] = u1
    v = jnp.dot(u1, w2_ref[...], preferred_element_type=jnp.float32)
    v0 = (wb_ref[0, 0] * v).astype(jnp.bfloat16)
    v1 = (wb_ref[1, 0] * v).astype(jnp.bfloat16)
    vs_stage[0, pl.ds(r * BM, BM), :] = v0
    vs_stage[1, pl.ds(r * BM, BM), :] = v1
    vs_out_ref[0] = v0
    vs_out_ref[1] = v1

    u2p_ref[...] = (
        jnp.dot(a0, vs_group[0], preferred_element_type=jnp.float32)
        + jnp.dot(a1, vs_group[1], preferred_element_type=jnp.float32)
    )

    @pl.when(r % RG == RG - 1)
    def _publish_group():
        g = r // RG
        vs_group[0, pl.ds(g * CW, CW), :] = vs_stage[0, pl.ds(g * CW, CW), :]
        vs_group[1, pl.ds(g * CW, CW), :] = vs_stage[1, pl.ds(g * CW, CW), :]


def _remainder_kernel(rmap_ref, cmap_ref, a_ref, vs_ref, u1_ref, u2p_ref,
                      b2_ref, out_ref, acc_ref):
    t = pl.program_id(0)
    r = rmap_ref[t]
    c = cmap_ref[t]

    @pl.when(c == r)
    def _init():
        acc_ref[...] = u2p_ref[...]

    acc_ref[...] += (
        jnp.dot(a_ref[0].astype(jnp.bfloat16), vs_ref[0],
                preferred_element_type=jnp.float32)
        + jnp.dot(a_ref[1].astype(jnp.bfloat16), vs_ref[1],
                  preferred_element_type=jnp.float32)
    )

    @pl.when(c == NC - 1)
    def _final():
        out_ref[...] = 0.5 * (u1_ref[...] + acc_ref[...] + b2_ref[...])


@jax.jit
def kernel(feature, A, W1, b1, W2, b2, weight_b):
    n = A.shape[1]
    hid = W1.shape[1]
    out_dim = W2.shape[1]

    zs = pl.pallas_call(
        _scaled_rhs_kernel,
        out_shape=jax.ShapeDtypeStruct((2, n, hid), jnp.bfloat16),
    )(feature, W1, weight_b)

    u1, u2p, vs = pl.pallas_call(
        _main_kernel,
        grid=(n // BM,),
        in_specs=[
            pl.BlockSpec((2, BM, n), lambda r: (0, r, 0)),
            pl.BlockSpec((2, n, hid), lambda r: (0, 0, 0)),
            pl.BlockSpec((hid, out_dim), lambda r: (0, 0)),
            pl.BlockSpec((2, 1), lambda r: (0, 0)),
            pl.BlockSpec((1, hid), lambda r: (0, 0)),
        ],
        out_specs=[
            pl.BlockSpec((BM, hid), lambda r: (r, 0)),
            pl.BlockSpec((BM, out_dim), lambda r: (r, 0)),
            pl.BlockSpec((2, BM, out_dim), lambda r: (0, r, 0)),
        ],
        out_shape=[
            jax.ShapeDtypeStruct((n, hid), jnp.float32),
            jax.ShapeDtypeStruct((n, out_dim), jnp.float32),
            jax.ShapeDtypeStruct((2, n, out_dim), jnp.bfloat16),
        ],
        scratch_shapes=[
            pltpu.VMEM((2, n, out_dim), jnp.bfloat16),
            pltpu.VMEM((2, n, out_dim), jnp.bfloat16),
        ],
    )(A, zs, W2, weight_b, b1.reshape(1, hid))

    nc = n // CW
    pairs = [(r, c) for r in range(nc) for c in range(r, nc)]
    rmap = jnp.asarray(np.array([p[0] for p in pairs], dtype=np.int32))
    cmap = jnp.asarray(np.array([p[1] for p in pairs], dtype=np.int32))

    grid_spec = pltpu.PrefetchScalarGridSpec(
        num_scalar_prefetch=2,
        grid=(len(pairs),),
        in_specs=[
            pl.BlockSpec((2, CW, CW), lambda t, rm, cm: (0, rm[t], cm[t])),
            pl.BlockSpec((2, CW, out_dim), lambda t, rm, cm: (0, cm[t], 0)),
            pl.BlockSpec((CW, hid), lambda t, rm, cm: (rm[t], 0)),
            pl.BlockSpec((CW, out_dim), lambda t, rm, cm: (rm[t], 0)),
            pl.BlockSpec((1, out_dim), lambda t, rm, cm: (0, 0)),
        ],
        out_specs=pl.BlockSpec((CW, out_dim), lambda t, rm, cm: (rm[t], 0)),
        scratch_shapes=[pltpu.VMEM((CW, out_dim), jnp.float32)],
    )

    out = pl.pallas_call(
        _remainder_kernel,
        grid_spec=grid_spec,
        out_shape=jax.ShapeDtypeStruct((n, out_dim), jnp.float32),
    )(rmap, cmap, A, vs, u1, u2p, b2.reshape(1, out_dim))

    return out


# speedup vs baseline: 1.1320x; 1.0610x over previous
import functools

import jax
import jax.numpy as jnp
import numpy as np
from jax.experimental import pallas as pl
from jax.experimental.pallas import tpu as pltpu

N = 4096
BM = 256
CW = 1024
NC = N // CW
RG = CW // BM


def _scaled_rhs_kernel(x_ref, w_ref, wb_ref, out_ref):
    z = jnp.dot(x_ref[...], w_ref[...], preferred_element_type=jnp.float32)
    out_ref[0] = (wb_ref[0, 0] * z).astype(jnp.bfloat16)
    out_ref[1] = (wb_ref[1, 0] * z).astype(jnp.bfloat16)


def _main_kernel(a_ref, zs_ref, w2_ref, wb_ref, b1_ref,
                 u1_ref, u2p_ref, vs_out_ref, vs_stage, rhs_scr):
    r = pl.program_id(0)
    hid = zs_ref.shape[2]

    @pl.when(r == 0)
    def _init_rhs():
        rhs_scr[0, :, :hid] = zs_ref[0]
        rhs_scr[1, :, :hid] = zs_ref[1]
        rhs_scr[0, :, hid:] = jnp.zeros_like(rhs_scr[0, :, hid:])
        rhs_scr[1, :, hid:] = jnp.zeros_like(rhs_scr[1, :, hid:])

    a0 = a_ref[0].astype(jnp.bfloat16)
    a1 = a_ref[1].astype(jnp.bfloat16)
    s = (
        jnp.dot(a0, rhs_scr[0], preferred_element_type=jnp.float32)
        + jnp.dot(a1, rhs_scr[1], preferred_element_type=jnp.float32)
    )
    u1 = jnp.maximum(s[:, :hid] + b1_ref[...], 0.0)
    u1_ref[...] = u1
    u2p_ref[...] = s[:, hid:]

    v = jnp.dot(u1, w2_ref[...], preferred_element_type=jnp.float32)
    v0 = (wb_ref[0, 0] * v).astype(jnp.bfloat16)
    v1 = (wb_ref[1, 0] * v).astype(jnp.bfloat16)
    vs_stage[0, pl.ds(r * BM, BM), :] = v0
    vs_stage[1, pl.ds(r * BM, BM), :] = v1
    vs_out_ref[0] = v0
    vs_out_ref[1] = v1

    @pl.when(r % RG == RG - 1)
    def _publish_group():
        g = r // RG
        rhs_scr[0, pl.ds(g * CW, CW), hid:] = vs_stage[0, pl.ds(g * CW, CW), :]
        rhs_scr[1, pl.ds(g * CW, CW), hid:] = vs_stage[1, pl.ds(g * CW, CW), :]


def _remainder_kernel(rmap_ref, cmap_ref, a_ref, vs_ref, u1_ref, u2p_ref,
                      b2_ref, out_ref, acc_ref):
    t = pl.program_id(0)
    r = rmap_ref[t]
    c = cmap_ref[t]

    @pl.when(c == r)
    def _init():
        acc_ref[...] = u2p_ref[...]

    acc_ref[...] += (
        jnp.dot(a_ref[0].astype(jnp.bfloat16), vs_ref[0],
                preferred_element_type=jnp.float32)
        + jnp.dot(a_ref[1].astype(jnp.bfloat16), vs_ref[1],
                  preferred_element_type=jnp.float32)
    )

    @pl.when(c == NC - 1)
    def _final():
        out_ref[...] = 0.5 * (u1_ref[...] + acc_ref[...] + b2_ref[...])


@jax.jit
def kernel(feature, A, W1, b1, W2, b2, weight_b):
    n = A.shape[1]
    hid = W1.shape[1]
    out_dim = W2.shape[1]

    zs = pl.pallas_call(
        _scaled_rhs_kernel,
        out_shape=jax.ShapeDtypeStruct((2, n, hid), jnp.bfloat16),
    )(feature, W1, weight_b)

    u1, u2p, vs = pl.pallas_call(
        _main_kernel,
        grid=(n // BM,),
        in_specs=[
            pl.BlockSpec((2, BM, n), lambda r: (0, r, 0)),
            pl.BlockSpec((2, n, hid), lambda r: (0, 0, 0)),
            pl.BlockSpec((hid, out_dim), lambda r: (0, 0)),
            pl.BlockSpec((2, 1), lambda r: (0, 0)),
            pl.BlockSpec((1, hid), lambda r: (0, 0)),
        ],
        out_specs=[
            pl.BlockSpec((BM, hid), lambda r: (r, 0)),
            pl.BlockSpec((BM, out_dim), lambda r: (r, 0)),
            pl.BlockSpec((2, BM, out_dim), lambda r: (0, r, 0)),
        ],
        out_shape=[
            jax.ShapeDtypeStruct((n, hid), jnp.float32),
            jax.ShapeDtypeStruct((n, out_dim), jnp.float32),
            jax.ShapeDtypeStruct((2, n, out_dim), jnp.bfloat16),
        ],
        scratch_shapes=[
            pltpu.VMEM((2, n, out_dim), jnp.bfloat16),
            pltpu.VMEM((2, n, hid + out_dim), jnp.bfloat16),
        ],
    )(A, zs, W2, weight_b, b1.reshape(1, hid))

    nc = n // CW
    pairs = [(r, c) for r in range(nc) for c in range(r, nc)]
    rmap = jnp.asarray(np.array([p[0] for p in pairs], dtype=np.int32))
    cmap = jnp.asarray(np.array([p[1] for p in pairs], dtype=np.int32))

    grid_spec = pltpu.PrefetchScalarGridSpec(
        num_scalar_prefetch=2,
        grid=(len(pairs),),
        in_specs=[
            pl.BlockSpec((2, CW, CW), lambda t, rm, cm: (0, rm[t], cm[t])),
            pl.BlockSpec((2, CW, out_dim), lambda t, rm, cm: (0, cm[t], 0)),
            pl.BlockSpec((CW, hid), lambda t, rm, cm: (rm[t], 0)),
            pl.BlockSpec((CW, out_dim), lambda t, rm, cm: (rm[t], 0)),
            pl.BlockSpec((1, out_dim), lambda t, rm, cm: (0, 0)),
        ],
        out_specs=pl.BlockSpec((CW, out_dim), lambda t, rm, cm: (rm[t], 0)),
        scratch_shapes=[pltpu.VMEM((CW, out_dim), jnp.float32)],
    )

    out = pl.pallas_call(
        _remainder_kernel,
        grid_spec=grid_spec,
        out_shape=jax.ShapeDtypeStruct((n, out_dim), jnp.float32),
    )(rmap, cmap, A, vs, u1, u2p, b2.reshape(1, out_dim))

    return out


# speedup vs baseline: 1.3369x; 1.1810x over previous
import functools

import jax
import jax.numpy as jnp
import numpy as np
from jax.experimental import pallas as pl
from jax.experimental.pallas import tpu as pltpu

N = 4096
BM = 256
CW = 1024
NC = N // CW
RG = CW // BM

_SLOT = {(1, 1): 0, (1, 2): 1, (1, 3): 2, (2, 2): 3, (2, 3): 4, (3, 3): 5}


def _scaled_rhs_kernel(x_ref, w_ref, wb_ref, out_ref):
    z = jnp.dot(x_ref[...], w_ref[...], preferred_element_type=jnp.float32)
    out_ref[0] = (wb_ref[0, 0] * z).astype(jnp.bfloat16)
    out_ref[1] = (wb_ref[1, 0] * z).astype(jnp.bfloat16)


def _fused_kernel(amr_ref, gmap_ref, cmap_ref, slot_ref, oa_idx_ref, ob_idx_ref,
                  a_ref, zs_ref, w2_ref, wb_ref, b1_ref, b2_ref,
                  oa_ref, ob_ref,
                  rhs_scr, vs_stage, u1_scr, u2p_scr, acc_scr, cache_scr):
    del oa_idx_ref, ob_idx_ref
    t = pl.program_id(0)
    r = amr_ref[t]
    g = gmap_ref[t]
    c = cmap_ref[t]
    sb = slot_ref[t]
    hid = zs_ref.shape[2]
    n_mega = N // BM

    @pl.when(t == 0)
    def _init_rhs():
        rhs_scr[0, :, :hid] = zs_ref[0]
        rhs_scr[1, :, :hid] = zs_ref[1]
        rhs_scr[0, :, hid:] = jnp.zeros_like(rhs_scr[0, :, hid:])
        rhs_scr[1, :, hid:] = jnp.zeros_like(rhs_scr[1, :, hid:])

    @pl.when(t < n_mega)
    def _mega():
        a0 = a_ref[0].astype(jnp.bfloat16)
        a1 = a_ref[1].astype(jnp.bfloat16)
        s = (
            jnp.dot(a0, rhs_scr[0], preferred_element_type=jnp.float32)
            + jnp.dot(a1, rhs_scr[1], preferred_element_type=jnp.float32)
        )
        u1 = jnp.maximum(s[:, :hid] + b1_ref[...], 0.0)
        u1_scr[pl.ds(r * BM, BM), :] = u1
        u2p_scr[pl.ds(r * BM, BM), :] = s[:, hid:]
        v = jnp.dot(u1, w2_ref[...], preferred_element_type=jnp.float32)
        vs_stage[0, pl.ds(r * BM, BM), :] = (wb_ref[0, 0] * v).astype(jnp.bfloat16)
        vs_stage[1, pl.ds(r * BM, BM), :] = (wb_ref[1, 0] * v).astype(jnp.bfloat16)

        @pl.when(r % RG == RG - 1)
        def _publish_group():
            gg = r // RG
            rhs_scr[0, pl.ds(gg * CW, CW), hid:] = vs_stage[0, pl.ds(gg * CW, CW), :]
            rhs_scr[1, pl.ds(gg * CW, CW), hid:] = vs_stage[1, pl.ds(gg * CW, CW), :]

        for gg in range(1, NC):
            @pl.when(r // RG == gg)
            def _cache(gg=gg, a0=a0, a1=a1):
                q = r % RG
                for cc in range(gg, NC):
                    slot_id = _SLOT[(gg, cc)]
                    sl = slice(cc * CW, (cc + 1) * CW)
                    cache_scr[slot_id, 0, pl.ds(q * BM, BM), :] = a0[:, sl]
                    cache_scr[slot_id, 1, pl.ds(q * BM, BM), :] = a1[:, sl]

    @pl.when((t >= n_mega) & (t < n_mega + RG))
    def _fresh_rows_g0():
        a0 = a_ref[0].astype(jnp.bfloat16)
        a1 = a_ref[1].astype(jnp.bfloat16)
        u2 = (
            jnp.dot(a0, vs_stage[0], preferred_element_type=jnp.float32)
            + jnp.dot(a1, vs_stage[1], preferred_element_type=jnp.float32)
        )
        oa_ref[...] = 0.5 * (u1_scr[pl.ds(r * BM, BM), :] + u2 + b2_ref[...])

    @pl.when(t >= n_mega + RG)
    def _cached_rows():
        @pl.when(c == g)
        def _init_acc():
            acc_scr[...] = u2p_scr[pl.ds(g * CW, CW), :]

        acc_scr[...] += (
            jnp.dot(cache_scr[sb, 0], vs_stage[0, pl.ds(c * CW, CW), :],
                    preferred_element_type=jnp.float32)
            + jnp.dot(cache_scr[sb, 1], vs_stage[1, pl.ds(c * CW, CW), :],
                      preferred_element_type=jnp.float32)
        )

        @pl.when(c == NC - 1)
        def _final():
            ob_ref[...] = 0.5 * (u1_scr[pl.ds(g * CW, CW), :]
                                 + acc_scr[...] + b2_ref[...])


@jax.jit
def kernel(feature, A, W1, b1, W2, b2, weight_b):
    n = A.shape[1]
    hid = W1.shape[1]
    out_dim = W2.shape[1]

    zs = pl.pallas_call(
        _scaled_rhs_kernel,
        out_shape=jax.ShapeDtypeStruct((2, n, hid), jnp.bfloat16),
    )(feature, W1, weight_b)

    n_mega = n // BM
    amr, gmap, cmap, slot, oa, ob = [], [], [], [], [], []
    for t in range(n_mega):
        amr.append(t); gmap.append(0); cmap.append(0); slot.append(0)
        oa.append(0); ob.append(0)
    for q in range(RG):
        amr.append(q); gmap.append(0); cmap.append(0); slot.append(0)
        oa.append(q); ob.append(0)
    for g in range(1, NC):
        for c in range(g, NC):
            amr.append(RG - 1); gmap.append(g); cmap.append(c)
            slot.append(_SLOT[(g, c)]); oa.append(RG - 1); ob.append(g - 1)
    T = len(amr)

    as_i32 = lambda xs: jnp.asarray(np.array(xs, dtype=np.int32))

    grid_spec = pltpu.PrefetchScalarGridSpec(
        num_scalar_prefetch=6,
        grid=(T,),
        in_specs=[
            pl.BlockSpec((2, BM, n),
                         lambda t, amr, gm, cm, sl, oa, ob: (0, amr[t], 0)),
            pl.BlockSpec((2, n, hid),
                         lambda t, amr, gm, cm, sl, oa, ob: (0, 0, 0)),
            pl.BlockSpec((hid, out_dim),
                         lambda t, amr, gm, cm, sl, oa, ob: (0, 0)),
            pl.BlockSpec((2, 1),
                         lambda t, amr, gm, cm, sl, oa, ob: (0, 0)),
            pl.BlockSpec((1, hid),
                         lambda t, amr, gm, cm, sl, oa, ob: (0, 0)),
            pl.BlockSpec((1, out_dim),
                         lambda t, amr, gm, cm, sl, oa, ob: (0, 0)),
        ],
        out_specs=[
            pl.BlockSpec((BM, out_dim),
                         lambda t, amr, gm, cm, sl, oa, ob: (oa[t], 0)),
            pl.BlockSpec((CW, out_dim),
                         lambda t, amr, gm, cm, sl, oa, ob: (ob[t], 0)),
        ],
        scratch_shapes=[
            pltpu.VMEM((2, n, hid + out_dim), jnp.bfloat16),
            pltpu.VMEM((2, n, out_dim), jnp.bfloat16),
            pltpu.VMEM((n, hid), jnp.float32),
            pltpu.VMEM((n, out_dim), jnp.float32),
            pltpu.VMEM((CW, out_dim), jnp.float32),
            pltpu.VMEM((len(_SLOT), 2, CW, CW), jnp.bfloat16),
        ],
    )

    out_a, out_b = pl.pallas_call(
        _fused_kernel,
        grid_spec=grid_spec,
        out_shape=[
            jax.ShapeDtypeStruct((CW, out_dim), jnp.float32),
            jax.ShapeDtypeStruct((n - CW, out_dim), jnp.float32),
        ],
    )(as_i32(amr), as_i32(gmap), as_i32(cmap), as_i32(slot),
      as_i32(oa), as_i32(ob),
      A, zs, W2, weight_b, b1.reshape(1, hid), b2.reshape(1, out_dim))

    return jnp.concatenate([out_a, out_b], axis=0)


# speedup vs baseline: 1.3967x; 1.0447x over previous
import functools

import jax
import jax.numpy as jnp
import numpy as np
from jax.experimental import pallas as pl
from jax.experimental.pallas import tpu as pltpu

N = 4096
BM = 256
CW = 1024
NC = N // CW
RG = CW // BM

_SLOT = {(1, 1): 0, (1, 2): 1, (1, 3): 2, (2, 2): 3, (2, 3): 4, (3, 3): 5}


def _fused_kernel(amr_ref, gmap_ref, cmap_ref, slot_ref, oa_idx_ref, ob_idx_ref,
                  a_ref, x_ref, w1_ref, w2_ref, wb_ref, b1_ref, b2_ref,
                  oa_ref, ob_ref,
                  rhs_scr, vs_stage, u1_scr, u2p_scr, acc_scr, cache_scr):
    del oa_idx_ref, ob_idx_ref
    t = pl.program_id(0)
    r = amr_ref[t]
    g = gmap_ref[t]
    c = cmap_ref[t]
    sb = slot_ref[t]
    hid = w1_ref.shape[1]
    n_mega = N // BM

    @pl.when(t == 0)
    def _init_rhs():
        z = jnp.dot(x_ref[...], w1_ref[...], preferred_element_type=jnp.float32)
        rhs_scr[0, :, :hid] = (wb_ref[0, 0] * z).astype(jnp.bfloat16)
        rhs_scr[1, :, :hid] = (wb_ref[1, 0] * z).astype(jnp.bfloat16)
        rhs_scr[0, :, hid:] = jnp.zeros_like(rhs_scr[0, :, hid:])
        rhs_scr[1, :, hid:] = jnp.zeros_like(rhs_scr[1, :, hid:])

    @pl.when(t < n_mega)
    def _mega():
        a0 = a_ref[0].astype(jnp.bfloat16)
        a1 = a_ref[1].astype(jnp.bfloat16)
        s = (
            jnp.dot(a0, rhs_scr[0], preferred_element_type=jnp.float32)
            + jnp.dot(a1, rhs_scr[1], preferred_element_type=jnp.float32)
        )
        u1 = jnp.maximum(s[:, :hid] + b1_ref[...], 0.0)
        u1_scr[pl.ds(r * BM, BM), :] = u1
        u2p_scr[pl.ds(r * BM, BM), :] = s[:, hid:]
        v = jnp.dot(u1, w2_ref[...], preferred_element_type=jnp.float32)
        vs_stage[0, pl.ds(r * BM, BM), :] = (wb_ref[0, 0] * v).astype(jnp.bfloat16)
        vs_stage[1, pl.ds(r * BM, BM), :] = (wb_ref[1, 0] * v).astype(jnp.bfloat16)

        @pl.when(r % RG == RG - 1)
        def _publish_group():
            gg = r // RG
            rhs_scr[0, pl.ds(gg * CW, CW), hid:] = vs_stage[0, pl.ds(gg * CW, CW), :]
            rhs_scr[1, pl.ds(gg * CW, CW), hid:] = vs_stage[1, pl.ds(gg * CW, CW), :]

        for gg in range(1, NC):
            @pl.when(r // RG == gg)
            def _cache(gg=gg, a0=a0, a1=a1):
                q = r % RG
                for cc in range(gg, NC):
                    slot_id = _SLOT[(gg, cc)]
                    sl = slice(cc * CW, (cc + 1) * CW)
                    cache_scr[slot_id, 0, pl.ds(q * BM, BM), :] = a0[:, sl]
                    cache_scr[slot_id, 1, pl.ds(q * BM, BM), :] = a1[:, sl]

    @pl.when((t >= n_mega) & (t < n_mega + RG))
    def _fresh_rows_g0():
        a0 = a_ref[0].astype(jnp.bfloat16)
        a1 = a_ref[1].astype(jnp.bfloat16)
        u2 = (
            jnp.dot(a0, vs_stage[0], preferred_element_type=jnp.float32)
            + jnp.dot(a1, vs_stage[1], preferred_element_type=jnp.float32)
        )
        oa_ref[...] = 0.5 * (u1_scr[pl.ds(r * BM, BM), :] + u2 + b2_ref[...])

    @pl.when(t >= n_mega + RG)
    def _cached_rows():
        @pl.when(c == g)
        def _init_acc():
            acc_scr[...] = u2p_scr[pl.ds(g * CW, CW), :]

        acc_scr[...] += (
            jnp.dot(cache_scr[sb, 0], vs_stage[0, pl.ds(c * CW, CW), :],
                    preferred_element_type=jnp.float32)
            + jnp.dot(cache_scr[sb, 1], vs_stage[1, pl.ds(c * CW, CW), :],
                      preferred_element_type=jnp.float32)
        )

        @pl.when(c == NC - 1)
        def _final():
            ob_ref[...] = 0.5 * (u1_scr[pl.ds(g * CW, CW), :]
                                 + acc_scr[...] + b2_ref[...])


@jax.jit
def kernel(feature, A, W1, b1, W2, b2, weight_b):
    n = A.shape[1]
    hid = W1.shape[1]
    out_dim = W2.shape[1]

    n_mega = n // BM
    amr, gmap, cmap, slot, oa, ob = [], [], [], [], [], []
    for t in range(n_mega):
        amr.append(t); gmap.append(0); cmap.append(0); slot.append(0)
        oa.append(0); ob.append(0)
    for q in range(RG):
        amr.append(q); gmap.append(0); cmap.append(0); slot.append(0)
        oa.append(q); ob.append(0)
    for g in range(1, NC):
        for c in range(g, NC):
            amr.append(RG - 1); gmap.append(g); cmap.append(c)
            slot.append(_SLOT[(g, c)]); oa.append(RG - 1); ob.append(g - 1)
    T = len(amr)

    as_i32 = lambda xs: jnp.asarray(np.array(xs, dtype=np.int32))

    grid_spec = pltpu.PrefetchScalarGridSpec(
        num_scalar_prefetch=6,
        grid=(T,),
        in_specs=[
            pl.BlockSpec((2, BM, n),
                         lambda t, amr, gm, cm, sl, oa, ob: (0, amr[t], 0)),
            pl.BlockSpec((n, feature.shape[1]),
                         lambda t, amr, gm, cm, sl, oa, ob: (0, 0)),
            pl.BlockSpec((feature.shape[1], hid),
                         lambda t, amr, gm, cm, sl, oa, ob: (0, 0)),
            pl.BlockSpec((hid, out_dim),
                         lambda t, amr, gm, cm, sl, oa, ob: (0, 0)),
            pl.BlockSpec((2, 1),
                         lambda t, amr, gm, cm, sl, oa, ob: (0, 0)),
            pl.BlockSpec((1, hid),
                         lambda t, amr, gm, cm, sl, oa, ob: (0, 0)),
            pl.BlockSpec((1, out_dim),
                         lambda t, amr, gm, cm, sl, oa, ob: (0, 0)),
        ],
        out_specs=[
            pl.BlockSpec((BM, out_dim),
                         lambda t, amr, gm, cm, sl, oa, ob: (oa[t], 0)),
            pl.BlockSpec((CW, out_dim),
                         lambda t, amr, gm, cm, sl, oa, ob: (ob[t], 0)),
        ],
        scratch_shapes=[
            pltpu.VMEM((2, n, hid + out_dim), jnp.bfloat16),
            pltpu.VMEM((2, n, out_dim), jnp.bfloat16),
            pltpu.VMEM((n, hid), jnp.float32),
            pltpu.VMEM((n, out_dim), jnp.float32),
            pltpu.VMEM((CW, out_dim), jnp.float32),
            pltpu.VMEM((len(_SLOT), 2, CW, CW), jnp.bfloat16),
        ],
    )

    out_a, out_b = pl.pallas_call(
        _fused_kernel,
        grid_spec=grid_spec,
        out_shape=[
            jax.ShapeDtypeStruct((CW, out_dim), jnp.float32),
            jax.ShapeDtypeStruct((n - CW, out_dim), jnp.float32),
        ],
    )(as_i32(amr), as_i32(gmap), as_i32(cmap), as_i32(slot),
      as_i32(oa), as_i32(ob),
      A, feature, W1, W2, weight_b, b1.reshape(1, hid), b2.reshape(1, out_dim))

    return jnp.concatenate([out_a, out_b], axis=0)
